# baseline (speedup 1.0000x reference)
import jax
import jax.numpy as jnp
from jax.experimental import pallas as pl

N_HIGH = 50000
EPS = 1e-5


def _batchnorm(x, g, b):
    m = x.mean(0)
    v = x.var(0)
    return (x - m) / jnp.sqrt(v + EPS) * g + b


def _gru(x, Wih, Whh, bih, bhh):
    def step(h, xt):
        gx = xt @ Wih.T + bih
        gh = h @ Whh.T + bhh
        xr, xz, xn = jnp.split(gx, 3, axis=-1)
        hr, hz, hn = jnp.split(gh, 3, axis=-1)
        r = jax.nn.sigmoid(xr + hr)
        z = jax.nn.sigmoid(xz + hz)
        n = jnp.tanh(xn + r * hn)
        h_new = (1.0 - z) * n + z * h
        return h_new, h_new
    h0 = jnp.zeros((x.shape[0], Whh.shape[1]), x.dtype)
    _, ys = jax.lax.scan(step, h0, jnp.swapaxes(x, 0, 1))
    return jnp.swapaxes(ys, 0, 1)


def _mean_aggr(msg, dst, N):
    s = jax.ops.segment_sum(msg, dst, num_segments=N)
    c = jax.ops.segment_sum(jnp.ones((msg.shape[0],), msg.dtype), dst, num_segments=N)
    return s / jnp.maximum(c, 1.0)[:, None]


def _gatv2(x, src, dst, Wl, Wr, att, bias, N):
    xl = x @ Wl.T
    xr = x @ Wr.T
    e = jax.nn.leaky_relu(xl[src] + xr[dst], 0.2) @ att
    m = jax.ops.segment_max(e, dst, num_segments=N)
    ee = jnp.exp(e - m[dst])
    den = jax.ops.segment_sum(ee, dst, num_segments=N)
    alpha = ee / jnp.maximum(den, 1e-16)[dst]
    msg = alpha[:, None] * xl[src]
    return _mean_aggr(msg, dst, N) + bias


def _pred_kernel(h_ref, w_ref, b_ref, o_ref):
    o_ref[...] = jnp.sum(h_ref[...] * w_ref[...], axis=1, keepdims=True) + b_ref[...]


def kernel(low_x, z_std, land_std, edge_src_l2h, edge_dst_l2h, edge_index_high,
           gru_Wih, gru_Whh, gru_bih, gru_bhh, dense_W, dense_b, dbn_g, dbn_b,
           gc_Wrel, gc_brel, gc_Wroot, gat_Wl, gat_Wr, gat_att, gat_b, bn_g, bn_b,
           pred_W, pred_b):
    enc = _gru(low_x, gru_Wih, gru_Whh, gru_bih, gru_bhh).reshape(low_x.shape[0], -1)
    enc = jnp.maximum(enc @ dense_W.T + dense_b, 0.0)
    enc = _batchnorm(enc, dbn_g, dbn_b)
    x_zland = jnp.concatenate([z_std, land_std], axis=-1)
    agg = _mean_aggr(enc[edge_src_l2h], edge_dst_l2h, N_HIGH)
    h = agg @ gc_Wrel.T + gc_brel + x_zland @ gc_Wroot.T
    sl = jnp.arange(N_HIGH)
    src = jnp.concatenate([edge_index_high[0], sl])
    dst = jnp.concatenate([edge_index_high[1], sl])
    h = _batchnorm(h, bn_g[0], bn_b[0])
    for i in range(4):
        h = _gatv2(h, src, dst, gat_Wl[i], gat_Wr[i], gat_att[i], gat_b[i], N_HIGH)
        h = jnp.maximum(_batchnorm(h, bn_g[i + 1], bn_b[i + 1]), 0.0)
    h = jnp.maximum(_gatv2(h, src, dst, gat_Wl[4], gat_Wr[4], gat_att[4], gat_b[4], N_HIGH), 0.0)
    out = pl.pallas_call(
        _pred_kernel,
        out_shape=jax.ShapeDtypeStruct((h.shape[0], 1), h.dtype),
    )(h, pred_W, pred_b)
    return out


# speedup vs baseline: 7.9424x; 7.9424x over previous
import jax
import jax.numpy as jnp
from jax import lax
from jax.experimental import pallas as pl
from jax.experimental.pallas import tpu as pltpu
from jax.experimental.pallas import tpu_sc as plsc

N_LOW = 10000
N_HIGH = 50000
E_L2H = 800000
E_HH = 1600000
EPS = 1e-5

NC = 2
NS = 16
NW = NC * NS
BLK = 128
KB = 8
NPAD = 50048
NPT = NPAD // NS

EPAD_HH = 1638400
EPAD_L2H = 819200

_mesh = plsc.VectorSubcoreMesh(core_axis_name="c", subcore_axis_name="s",
                               num_cores=NC, num_subcores=NS)
_cparams = pltpu.CompilerParams(use_tc_tiling_on_sc=False)


def _zero_accs(zeros16, zeros8, acc, acc2, s):
    pltpu.sync_copy(zeros16.at[pl.ds(s * NPT, NPT)], acc.at[pl.ds(s * NPT, NPT)])
    pltpu.sync_copy(zeros8.at[pl.ds(s * NPT, NPT)], acc2.at[pl.ds(s * NPT, NPT)])


def _dump_accs(acc, acc2, out, out2, c, s):
    pltpu.sync_copy(acc.at[pl.ds(s * NPT, NPT)], out.at[c, pl.ds(s * NPT, NPT)])
    pltpu.sync_copy(acc2.at[pl.ds(s * NPT, NPT)], out2.at[c, pl.ds(s * NPT, NPT)])


def _l2h_body(table, srcb, dstb, ones8, zeros16, zeros8, out, out2,
              acc, acc2, idxs, idxd, gbuf, obuf, gsem, ssem):
    c = lax.axis_index("c")
    s = lax.axis_index("s")
    w = s * NC + c
    bpw = EPAD_L2H // BLK // NW
    _zero_accs(zeros16, zeros8, acc, acc2, s)
    pltpu.sync_copy(ones8, obuf)
    plsc.subcore_barrier()

    def step(i, carry):
        base = w * bpw + i * KB
        pltpu.sync_copy(srcb.at[pl.ds(base, KB)], idxs)
        pltpu.sync_copy(dstb.at[pl.ds(base, KB)], idxd)
        gds = [pltpu.async_copy(table.at[idxs.at[j]], gbuf.at[j], gsem)
               for j in range(KB)]
        for d in gds:
            d.wait()
        sds = [pltpu.async_copy(gbuf.at[j], acc.at[idxd.at[j]], ssem, add=True)
               for j in range(KB)]
        sds += [pltpu.async_copy(obuf, acc2.at[idxd.at[j]], ssem, add=True)
                for j in range(KB)]
        for d in sds:
            d.wait()
        return carry

    lax.fori_loop(0, bpw // KB, step, 0)
    plsc.subcore_barrier()
    _dump_accs(acc, acc2, out, out2, c, s)


def _gat_gather_body(xl, xr, srcb, dstb, g1, g2, idxs, idxd, bufa, bufb, sem):
    c = lax.axis_index("c")
    s = lax.axis_index("s")
    w = s * NC + c
    bpw = EPAD_HH // BLK // NW

    def step(i, carry):
        base = w * bpw + i * KB
        pltpu.sync_copy(srcb.at[pl.ds(base, KB)], idxs)
        pltpu.sync_copy(dstb.at[pl.ds(base, KB)], idxd)
        ds = ([pltpu.async_copy(xl.at[idxs.at[j]], bufa.at[j], sem)
               for j in range(KB)] +
              [pltpu.async_copy(xr.at[idxd.at[j]], bufb.at[j], sem)
               for j in range(KB)])
        for d in ds:
            d.wait()
        pltpu.sync_copy(bufa, g1.at[pl.ds(base, KB)])
        pltpu.sync_copy(bufb, g2.at[pl.ds(base, KB)])
        return carry

    lax.fori_loop(0, bpw // KB, step, 0)


def _gat_scatter_body(m16, m8, dstb, zeros16, zeros8, out, out2,
                      acc, acc2, idxd, vbuf, vbuf2, gsem, ssem):
    c = lax.axis_index("c")
    s = lax.axis_index("s")
    w = s * NC + c
    bpw = EPAD_HH // BLK // NW
    _zero_accs(zeros16, zeros8, acc, acc2, s)
    plsc.subcore_barrier()

    def step(i, carry):
        base = w * bpw + i * KB
        pltpu.sync_copy(dstb.at[pl.ds(base, KB)], idxd)
        pltpu.sync_copy(m16.at[pl.ds(base, KB)], vbuf)
        pltpu.sync_copy(m8.at[pl.ds(base, KB)], vbuf2)
        sds = [pltpu.async_copy(vbuf.at[j], acc.at[idxd.at[j]], ssem, add=True)
               for j in range(KB)]
        sds += [pltpu.async_copy(vbuf2.at[j], acc2.at[idxd.at[j]], ssem, add=True)
                for j in range(KB)]
        for d in sds:
            d.wait()
        return carry

    lax.fori_loop(0, bpw // KB, step, 0)
    plsc.subcore_barrier()
    _dump_accs(acc, acc2, out, out2, c, s)


_l2h_call = pl.kernel(
    _l2h_body,
    out_type=[jax.ShapeDtypeStruct((NC, NPAD, 16), jnp.float32),
              jax.ShapeDtypeStruct((NC, NPAD, 8), jnp.float32)],
    mesh=_mesh,
    compiler_params=_cparams,
    scratch_types=[
        pltpu.VMEM_SHARED((NPAD, 16), jnp.float32),
        pltpu.VMEM_SHARED((NPAD, 8), jnp.float32),
        pltpu.VMEM((KB, BLK), jnp.int32),
        pltpu.VMEM((KB, BLK), jnp.int32),
        pltpu.VMEM((KB, BLK, 16), jnp.float32),
        pltpu.VMEM((BLK, 8), jnp.float32),
        pltpu.SemaphoreType.DMA,
        pltpu.SemaphoreType.DMA,
    ],
)

_gat_gather_call = pl.kernel(
    _gat_gather_body,
    out_type=[jax.ShapeDtypeStruct((EPAD_HH // BLK, BLK, 16), jnp.float32),
              jax.ShapeDtypeStruct((EPAD_HH // BLK, BLK, 16), jnp.float32)],
    mesh=_mesh,
    compiler_params=_cparams,
    scratch_types=[
        pltpu.VMEM((KB, BLK), jnp.int32),
        pltpu.VMEM((KB, BLK), jnp.int32),
        pltpu.VMEM((KB, BLK, 16), jnp.float32),
        pltpu.VMEM((KB, BLK, 16), jnp.float32),
        pltpu.SemaphoreType.DMA,
    ],
)

_gat_scatter_call = pl.kernel(
    _gat_scatter_body,
    out_type=[jax.ShapeDtypeStruct((NC, NPAD, 16), jnp.float32),
              jax.ShapeDtypeStruct((NC, NPAD, 8), jnp.float32)],
    mesh=_mesh,
    compiler_params=_cparams,
    scratch_types=[
        pltpu.VMEM_SHARED((NPAD, 16), jnp.float32),
        pltpu.VMEM_SHARED((NPAD, 8), jnp.float32),
        pltpu.VMEM((KB, BLK), jnp.int32),
        pltpu.VMEM((KB, BLK, 16), jnp.float32),
        pltpu.VMEM((KB, BLK, 8), jnp.float32),
        pltpu.SemaphoreType.DMA,
        pltpu.SemaphoreType.DMA,
    ],
)


def _batchnorm(x, g, b):
    m = x.mean(0)
    v = x.var(0)
    return (x - m) / jnp.sqrt(v + EPS) * g + b


def _gru(x, Wih, Whh, bih, bhh):
    def step(h, xt):
        gx = xt @ Wih.T + bih
        gh = h @ Whh.T + bhh
        xr, xz, xn = jnp.split(gx, 3, axis=-1)
        hr, hz, hn = jnp.split(gh, 3, axis=-1)
        r = jax.nn.sigmoid(xr + hr)
        z = jax.nn.sigmoid(xz + hz)
        n = jnp.tanh(xn + r * hn)
        h_new = (1.0 - z) * n + z * h
        return h_new, h_new
    h0 = jnp.zeros((x.shape[0], Whh.shape[1]), x.dtype)
    _, ys = jax.lax.scan(step, h0, jnp.swapaxes(x, 0, 1))
    return jnp.swapaxes(ys, 0, 1)


def _pad_edges(idx, epad, fill):
    pad = jnp.full((epad - idx.shape[0],), fill, jnp.int32)
    return jnp.concatenate([idx.astype(jnp.int32), pad]).reshape(-1, BLK)


def kernel(low_x, z_std, land_std, edge_src_l2h, edge_dst_l2h, edge_index_high,
           gru_Wih, gru_Whh, gru_bih, gru_bhh, dense_W, dense_b, dbn_g, dbn_b,
           gc_Wrel, gc_brel, gc_Wroot, gat_Wl, gat_Wr, gat_att, gat_b, bn_g, bn_b,
           pred_W, pred_b):
    f32 = jnp.float32
    zeros16 = jnp.zeros((NPAD, 16), f32)
    zeros8 = jnp.zeros((NPAD, 8), f32)
    ones8 = jnp.concatenate(
        [jnp.ones((BLK, 1), f32), jnp.zeros((BLK, 7), f32)], axis=1)

    enc = _gru(low_x, gru_Wih, gru_Whh, gru_bih, gru_bhh).reshape(N_LOW, -1)
    enc = jnp.maximum(enc @ dense_W.T + dense_b, 0.0)
    enc = _batchnorm(enc, dbn_g, dbn_b)

    encw = enc @ gc_Wrel.T
    srcb_l = _pad_edges(edge_src_l2h, EPAD_L2H, 0)
    dstb_l = _pad_edges(edge_dst_l2h, EPAD_L2H, N_HIGH)
    part16, part8 = _l2h_call(encw, srcb_l, dstb_l, ones8, zeros16, zeros8)
    tot = (part16[0] + part16[1])[:N_HIGH]
    cnt = (part8[0] + part8[1])[:N_HIGH, 0]
    x_zland = jnp.concatenate([z_std, land_std], axis=-1)
    h = (tot / jnp.maximum(cnt, 1.0)[:, None]
         + gc_brel + x_zland @ gc_Wroot.T)

    srcb = _pad_edges(edge_index_high[0], EPAD_HH, 0)
    dstb = _pad_edges(edge_index_high[1], EPAD_HH, N_HIGH)
    pad16 = jnp.zeros((NPAD - N_HIGH, 16), f32)
    h = _batchnorm(h, bn_g[0], bn_b[0])
    for i in range(5):
        xl = h @ gat_Wl[i].T
        xr = h @ gat_Wr[i].T
        att = gat_att[i]
        g1, g2 = _gat_gather_call(jnp.concatenate([xl, pad16]),
                                  jnp.concatenate([xr, pad16]), srcb, dstb)
        sv = g1 + g2
        t = jnp.maximum(sv, 0.2 * sv)
        ee = jnp.exp(t @ att)
        m16 = ee[..., None] * g1
        m8 = jnp.concatenate(
            [ee[..., None], jnp.ones_like(ee)[..., None],
             jnp.zeros(ee.shape + (6,), f32)], axis=-1)
        part16, part8 = _gat_scatter_call(m16, m8, dstb, zeros16, zeros8)
        tot = (part16[0] + part16[1])[:N_HIGH]
        aux = (part8[0] + part8[1])[:N_HIGH]
        ss = xl + xr
        ts = jnp.maximum(ss, 0.2 * ss)
        ees = jnp.exp(ts @ att)
        accv = tot + ees[:, None] * xl
        den = aux[:, 0] + ees
        cnt = aux[:, 1] + 1.0
        agg = accv / jnp.maximum(den, 1e-16)[:, None] / jnp.maximum(cnt, 1.0)[:, None]
        h = agg + gat_b[i]
        if i < 4:
            h = jnp.maximum(_batchnorm(h, bn_g[i + 1], bn_b[i + 1]), 0.0)
        else:
            h = jnp.maximum(h, 0.0)

    return h @ pred_W.T + pred_b


# speedup vs baseline: 27.7624x; 3.4955x over previous
import jax
import jax.numpy as jnp
from jax import lax
from jax.experimental import pallas as pl
from jax.experimental.pallas import tpu as pltpu
from jax.experimental.pallas import tpu_sc as plsc

N_LOW = 10000
N_HIGH = 50000
E_L2H = 800000
E_HH = 1600000
EPS = 1e-5

NC = 2
NS = 16
NW = NC * NS
BLK = 128
KB = 8
NPAD = 50048
NPT = NPAD // NS

EPAD_HH = 1638400
EPAD_L2H = 819200

_mesh = plsc.VectorSubcoreMesh(core_axis_name="c", subcore_axis_name="s",
                               num_cores=NC, num_subcores=NS)
_cparams = pltpu.CompilerParams(use_tc_tiling_on_sc=False, needs_layout_passes=False)


def _zero_accs(zeros16, zeros8, acc, acc2, s):
    pltpu.sync_copy(zeros16.at[pl.ds(s * NPT, NPT)], acc.at[pl.ds(s * NPT, NPT)])
    pltpu.sync_copy(zeros8.at[pl.ds(s * NPT, NPT)], acc2.at[pl.ds(s * NPT, NPT)])


def _dump_accs(acc, acc2, out, out2, c, s):
    pltpu.sync_copy(acc.at[pl.ds(s * NPT, NPT)], out.at[c, pl.ds(s * NPT, NPT)])
    pltpu.sync_copy(acc2.at[pl.ds(s * NPT, NPT)], out2.at[c, pl.ds(s * NPT, NPT)])


def _l2h_body(table, srcb, dstb, ones8, zeros16, zeros8, out, out2,
              acc, acc2, idxs, idxd, gbuf, obuf, gsem, ssem):
    c = lax.axis_index("c")
    s = lax.axis_index("s")
    w = s * NC + c
    bpw = EPAD_L2H // BLK // NW
    _zero_accs(zeros16, zeros8, acc, acc2, s)
    pltpu.sync_copy(ones8, obuf)
    plsc.subcore_barrier()

    def step(i, carry):
        base = w * bpw + i * KB
        pltpu.sync_copy(srcb.at[pl.ds(base, KB)], idxs)
        pltpu.sync_copy(dstb.at[pl.ds(base, KB)], idxd)
        gds = [pltpu.async_copy(table.at[idxs.at[j]], gbuf.at[j], gsem)
               for j in range(KB)]
        for d in gds:
            d.wait()
        sds = [pltpu.async_copy(gbuf.at[j], acc.at[idxd.at[j]], ssem, add=True)
               for j in range(KB)]
        sds += [pltpu.async_copy(obuf, acc2.at[idxd.at[j]], ssem, add=True)
                for j in range(KB)]
        for d in sds:
            d.wait()
        return carry

    lax.fori_loop(0, bpw // KB, step, 0)
    plsc.subcore_barrier()
    _dump_accs(acc, acc2, out, out2, c, s)


def _gat_fused_body(xl, xr, srcb, dstb, att, m8init, zeros16, zeros8,
                    out, out2, acc, acc2, idxs, idxd, bufa, bufb, m8buf,
                    attbuf, gsem, ssem):
    c = lax.axis_index("c")
    s = lax.axis_index("s")
    w = s * NC + c
    bpw = EPAD_HH // BLK // NW
    _zero_accs(zeros16, zeros8, acc, acc2, s)
    pltpu.sync_copy(att, attbuf)
    pltpu.sync_copy(m8init, m8buf)
    plsc.subcore_barrier()
    iota16 = lax.iota(jnp.int32, 16)
    attv = attbuf[...]
    att_s = [attv[k] for k in range(16)]
    kf = [jnp.full((16,), k, jnp.int32) for k in range(16)]
    zci = jnp.zeros((16,), jnp.int32)

    def grp(g, cc):
        lanes = g * 16 + iota16
        a_ks = []
        accv = jnp.zeros((16,), jnp.float32)
        for k in range(16):
            a = plsc.load_gather(bufa, [lanes, kf[k]])
            b = plsc.load_gather(bufb, [lanes, kf[k]])
            a_ks.append(a)
            sk = a + b
            tk = jnp.maximum(sk, 0.2 * sk)
            accv = accv + tk * att_s[k]
        ee = jnp.exp(accv)
        plsc.store_scatter(m8buf, [lanes, zci], ee)
        for k in range(16):
            plsc.store_scatter(bufa, [lanes, kf[k]], ee * a_ks[k])
        return cc

    def step(i, carry):
        base = w * bpw + i * KB
        pltpu.sync_copy(srcb.at[pl.ds(base, KB)], idxs)
        pltpu.sync_copy(dstb.at[pl.ds(base, KB)], idxd)
        gds = ([pltpu.async_copy(xl.at[idxs.at[j]],
                                 bufa.at[pl.ds(j * BLK, BLK)], gsem)
                for j in range(KB)] +
               [pltpu.async_copy(xr.at[idxd.at[j]],
                                 bufb.at[pl.ds(j * BLK, BLK)], gsem)
                for j in range(KB)])
        for d in gds:
            d.wait()
        lax.fori_loop(0, KB * 8, grp, 0)
        sds = ([pltpu.async_copy(bufa.at[pl.ds(j * BLK, BLK)],
                                 acc.at[idxd.at[j]], ssem, add=True)
                for j in range(KB)] +
               [pltpu.async_copy(m8buf.at[pl.ds(j * BLK, BLK)],
                                 acc2.at[idxd.at[j]], ssem, add=True)
                for j in range(KB)])
        for d in sds:
            d.wait()
        return carry

    lax.fori_loop(0, bpw // KB, step, 0)
    plsc.subcore_barrier()
    _dump_accs(acc, acc2, out, out2, c, s)


_l2h_call = pl.kernel(
    _l2h_body,
    out_type=[jax.ShapeDtypeStruct((NC, NPAD, 16), jnp.float32),
              jax.ShapeDtypeStruct((NC, NPAD, 8), jnp.float32)],
    mesh=_mesh,
    compiler_params=_cparams,
    scratch_types=[
        pltpu.VMEM_SHARED((NPAD, 16), jnp.float32),
        pltpu.VMEM_SHARED((NPAD, 8), jnp.float32),
        pltpu.VMEM((KB, BLK), jnp.int32),
        pltpu.VMEM((KB, BLK), jnp.int32),
        pltpu.VMEM((KB, BLK, 16), jnp.float32),
        pltpu.VMEM((BLK, 8), jnp.float32),
        pltpu.SemaphoreType.DMA,
        pltpu.SemaphoreType.DMA,
    ],
)

_gat_fused_call = pl.kernel(
    _gat_fused_body,
    out_type=[jax.ShapeDtypeStruct((NC, NPAD, 16), jnp.float32),
              jax.ShapeDtypeStruct((NC, NPAD, 8), jnp.float32)],
    mesh=_mesh,
    compiler_params=_cparams,
    scratch_types=[
        pltpu.VMEM_SHARED((NPAD, 16), jnp.float32),
        pltpu.VMEM_SHARED((NPAD, 8), jnp.float32),
        pltpu.VMEM((KB, BLK), jnp.int32),
        pltpu.VMEM((KB, BLK), jnp.int32),
        pltpu.VMEM((KB * BLK, 16), jnp.float32),
        pltpu.VMEM((KB * BLK, 16), jnp.float32),
        pltpu.VMEM((KB * BLK, 8), jnp.float32),
        pltpu.VMEM((16,), jnp.float32),
        pltpu.SemaphoreType.DMA,
        pltpu.SemaphoreType.DMA,
    ],
)


def _batchnorm(x, g, b):
    m = x.mean(0)
    v = x.var(0)
    return (x - m) / jnp.sqrt(v + EPS) * g + b


def _gru(x, Wih, Whh, bih, bhh):
    def step(h, xt):
        gx = xt @ Wih.T + bih
        gh = h @ Whh.T + bhh
        xr, xz, xn = jnp.split(gx, 3, axis=-1)
        hr, hz, hn = jnp.split(gh, 3, axis=-1)
        r = jax.nn.sigmoid(xr + hr)
        z = jax.nn.sigmoid(xz + hz)
        n = jnp.tanh(xn + r * hn)
        h_new = (1.0 - z) * n + z * h
        return h_new, h_new
    h0 = jnp.zeros((x.shape[0], Whh.shape[1]), x.dtype)
    _, ys = jax.lax.scan(step, h0, jnp.swapaxes(x, 0, 1))
    return jnp.swapaxes(ys, 0, 1)


def _pad_edges(idx, epad, fill):
    pad = jnp.full((epad - idx.shape[0],), fill, jnp.int32)
    return jnp.concatenate([idx.astype(jnp.int32), pad]).reshape(-1, BLK)


def kernel(low_x, z_std, land_std, edge_src_l2h, edge_dst_l2h, edge_index_high,
           gru_Wih, gru_Whh, gru_bih, gru_bhh, dense_W, dense_b, dbn_g, dbn_b,
           gc_Wrel, gc_brel, gc_Wroot, gat_Wl, gat_Wr, gat_att, gat_b, bn_g, bn_b,
           pred_W, pred_b):
    f32 = jnp.float32
    zeros16 = jnp.zeros((NPAD, 16), f32)
    zeros8 = jnp.zeros((NPAD, 8), f32)
    ones8 = jnp.concatenate(
        [jnp.ones((BLK, 1), f32), jnp.zeros((BLK, 7), f32)], axis=1)

    enc = _gru(low_x, gru_Wih, gru_Whh, gru_bih, gru_bhh).reshape(N_LOW, -1)
    enc = jnp.maximum(enc @ dense_W.T + dense_b, 0.0)
    enc = _batchnorm(enc, dbn_g, dbn_b)

    encw = enc @ gc_Wrel.T
    srcb_l = _pad_edges(edge_src_l2h, EPAD_L2H, 0)
    dstb_l = _pad_edges(edge_dst_l2h, EPAD_L2H, N_HIGH)
    part16, part8 = _l2h_call(encw, srcb_l, dstb_l, ones8, zeros16, zeros8)
    tot = (part16[0] + part16[1])[:N_HIGH]
    cnt = (part8[0] + part8[1])[:N_HIGH, 0]
    x_zland = jnp.concatenate([z_std, land_std], axis=-1)
    h = (tot / jnp.maximum(cnt, 1.0)[:, None]
         + gc_brel + x_zland @ gc_Wroot.T)

    srcb = _pad_edges(edge_index_high[0], EPAD_HH, 0)
    dstb = _pad_edges(edge_index_high[1], EPAD_HH, N_HIGH)
    pad16 = jnp.zeros((NPAD - N_HIGH, 16), f32)
    m8init = jnp.concatenate(
        [jnp.zeros((KB * BLK, 1), f32), jnp.ones((KB * BLK, 1), f32),
         jnp.zeros((KB * BLK, 6), f32)], axis=1)
    h = _batchnorm(h, bn_g[0], bn_b[0])
    for i in range(5):
        xl = h @ gat_Wl[i].T
        xr = h @ gat_Wr[i].T
        att = gat_att[i]
        part16, part8 = _gat_fused_call(
            jnp.concatenate([xl, pad16]), jnp.concatenate([xr, pad16]),
            srcb, dstb, att, m8init, zeros16, zeros8)
        tot = (part16[0] + part16[1])[:N_HIGH]
        aux = (part8[0] + part8[1])[:N_HIGH]
        ss = xl + xr
        ts = jnp.maximum(ss, 0.2 * ss)
        ees = jnp.exp(ts @ att)
        accv = tot + ees[:, None] * xl
        den = aux[:, 0] + ees
        cnt = aux[:, 1] + 1.0
        agg = accv / jnp.maximum(den, 1e-16)[:, None] / jnp.maximum(cnt, 1.0)[:, None]
        h = agg + gat_b[i]
        if i < 4:
            h = jnp.maximum(_batchnorm(h, bn_g[i + 1], bn_b[i + 1]), 0.0)
        else:
            h = jnp.maximum(h, 0.0)

    return h @ pred_W.T + pred_b


# speedup vs baseline: 39.6027x; 1.4265x over previous
import jax
import jax.numpy as jnp
from jax import lax
from jax.experimental import pallas as pl
from jax.experimental.pallas import tpu as pltpu
from jax.experimental.pallas import tpu_sc as plsc

N_LOW = 10000
N_HIGH = 50000
E_L2H = 800000
E_HH = 1600000
EPS = 1e-5

NC = 2
NS = 16
NW = NC * NS
BLK = 128
KB = 8
NPAD = 50048
NPT = NPAD // NS

EPAD_HH = 1638400
EPAD_L2H = 819200

_mesh = plsc.VectorSubcoreMesh(core_axis_name="c", subcore_axis_name="s",
                               num_cores=NC, num_subcores=NS)
_cparams = pltpu.CompilerParams(use_tc_tiling_on_sc=False, needs_layout_passes=False)


def _zero_accs(zeros16, zeros8, acc, acc2, s):
    pltpu.sync_copy(zeros16.at[pl.ds(s * NPT, NPT)], acc.at[pl.ds(s * NPT, NPT)])
    pltpu.sync_copy(zeros8.at[pl.ds(s * NPT, NPT)], acc2.at[pl.ds(s * NPT, NPT)])


def _dump_accs(acc, acc2, out, out2, c, s):
    pltpu.sync_copy(acc.at[pl.ds(s * NPT, NPT)], out.at[c, pl.ds(s * NPT, NPT)])
    pltpu.sync_copy(acc2.at[pl.ds(s * NPT, NPT)], out2.at[c, pl.ds(s * NPT, NPT)])


def _l2h_body(table, srcb, dstb, ones8, zeros16, zeros8, out, out2,
              acc, acc2, idxs, idxd, gbuf, obuf, gsem, ssem):
    c = lax.axis_index("c")
    s = lax.axis_index("s")
    w = s * NC + c
    bpw = EPAD_L2H // BLK // NW
    _zero_accs(zeros16, zeros8, acc, acc2, s)
    pltpu.sync_copy(ones8, obuf)
    plsc.subcore_barrier()

    def step(i, carry):
        base = w * bpw + i * KB
        pltpu.sync_copy(srcb.at[pl.ds(base, KB)], idxs)
        pltpu.sync_copy(dstb.at[pl.ds(base, KB)], idxd)
        gds = [pltpu.async_copy(table.at[idxs.at[j]], gbuf.at[j], gsem)
               for j in range(KB)]
        for d in gds:
            d.wait()
        sds = [pltpu.async_copy(gbuf.at[j], acc.at[idxd.at[j]], ssem, add=True)
               for j in range(KB)]
        sds += [pltpu.async_copy(obuf, acc2.at[idxd.at[j]], ssem, add=True)
                for j in range(KB)]
        for d in sds:
            d.wait()
        return carry

    lax.fori_loop(0, bpw // KB, step, 0)
    plsc.subcore_barrier()
    _dump_accs(acc, acc2, out, out2, c, s)


def _gat_fused_body(xl, xr, srcb, dstb, att, m8init, zeros16, zeros8,
                    out, out2, acc, acc2, idxs, idxd, bufa, bufb, m8buf,
                    attbuf, gsem, ssem):
    c = lax.axis_index("c")
    s = lax.axis_index("s")
    w = s * NC + c
    bpw = EPAD_HH // BLK // NW
    _zero_accs(zeros16, zeros8, acc, acc2, s)
    pltpu.sync_copy(att, attbuf)
    pltpu.sync_copy(m8init, m8buf)
    plsc.subcore_barrier()
    iota16 = lax.iota(jnp.int32, 16)
    attv = attbuf[...]
    att_s = [attv[k] for k in range(16)]
    kf = [jnp.full((16,), k, jnp.int32) for k in range(16)]
    zci = jnp.zeros((16,), jnp.int32)

    def grp(g, cc):
        lanes = g * 16 + iota16
        a_ks = []
        p0 = jnp.zeros((16,), jnp.float32)
        p1 = jnp.zeros((16,), jnp.float32)
        p2 = jnp.zeros((16,), jnp.float32)
        p3 = jnp.zeros((16,), jnp.float32)
        parts = [p0, p1, p2, p3]
        for k in range(16):
            a = plsc.load_gather(bufa, [lanes, kf[k]])
            b = plsc.load_gather(bufb, [lanes, kf[k]])
            a_ks.append(a)
            sk = a + b
            tk = jnp.maximum(sk, 0.2 * sk)
            parts[k % 4] = parts[k % 4] + tk * att_s[k]
        ee = jnp.exp((parts[0] + parts[1]) + (parts[2] + parts[3]))
        plsc.store_scatter(m8buf, [lanes, zci], ee)
        for k in range(16):
            plsc.store_scatter(bufa, [lanes, kf[k]], ee * a_ks[k])
        return cc

    def step(i, carry):
        base = w * bpw + i * KB
        pltpu.sync_copy(srcb.at[pl.ds(base, KB)], idxs)
        pltpu.sync_copy(dstb.at[pl.ds(base, KB)], idxd)
        gds = [(pltpu.async_copy(xl.at[idxs.at[j]],
                                 bufa.at[pl.ds(j * BLK, BLK)], gsem),
                pltpu.async_copy(xr.at[idxd.at[j]],
                                 bufb.at[pl.ds(j * BLK, BLK)], gsem))
               for j in range(KB)]
        sds = []
        for j in range(KB):
            gds[j][0].wait()
            gds[j][1].wait()
            lax.fori_loop(j * 8, (j + 1) * 8, grp, 0)
            sds.append(pltpu.async_copy(bufa.at[pl.ds(j * BLK, BLK)],
                                        acc.at[idxd.at[j]], ssem, add=True))
            sds.append(pltpu.async_copy(m8buf.at[pl.ds(j * BLK, BLK)],
                                        acc2.at[idxd.at[j]], ssem, add=True))
        for d in sds:
            d.wait()
        return carry

    lax.fori_loop(0, bpw // KB, step, 0)
    plsc.subcore_barrier()
    _dump_accs(acc, acc2, out, out2, c, s)


_l2h_call = pl.kernel(
    _l2h_body,
    out_type=[jax.ShapeDtypeStruct((NC, NPAD, 16), jnp.float32),
              jax.ShapeDtypeStruct((NC, NPAD, 8), jnp.float32)],
    mesh=_mesh,
    compiler_params=_cparams,
    scratch_types=[
        pltpu.VMEM_SHARED((NPAD, 16), jnp.float32),
        pltpu.VMEM_SHARED((NPAD, 8), jnp.float32),
        pltpu.VMEM((KB, BLK), jnp.int32),
        pltpu.VMEM((KB, BLK), jnp.int32),
        pltpu.VMEM((KB, BLK, 16), jnp.float32),
        pltpu.VMEM((BLK, 8), jnp.float32),
        pltpu.SemaphoreType.DMA,
        pltpu.SemaphoreType.DMA,
    ],
)

_gat_fused_call = pl.kernel(
    _gat_fused_body,
    out_type=[jax.ShapeDtypeStruct((NC, NPAD, 16), jnp.float32),
              jax.ShapeDtypeStruct((NC, NPAD, 8), jnp.float32)],
    mesh=_mesh,
    compiler_params=_cparams,
    scratch_types=[
        pltpu.VMEM_SHARED((NPAD, 16), jnp.float32),
        pltpu.VMEM_SHARED((NPAD, 8), jnp.float32),
        pltpu.VMEM((KB, BLK), jnp.int32),
        pltpu.VMEM((KB, BLK), jnp.int32),
        pltpu.VMEM((KB * BLK, 16), jnp.float32),
        pltpu.VMEM((KB * BLK, 16), jnp.float32),
        pltpu.VMEM((KB * BLK, 8), jnp.float32),
        pltpu.VMEM((16,), jnp.float32),
        pltpu.SemaphoreType.DMA,
        pltpu.SemaphoreType.DMA,
    ],
)


def _batchnorm(x, g, b):
    m = x.mean(0)
    v = x.var(0)
    return (x - m) / jnp.sqrt(v + EPS) * g + b


def _gru(x, Wih, Whh, bih, bhh):
    def step(h, xt):
        gx = xt @ Wih.T + bih
        gh = h @ Whh.T + bhh
        xr, xz, xn = jnp.split(gx, 3, axis=-1)
        hr, hz, hn = jnp.split(gh, 3, axis=-1)
        r = jax.nn.sigmoid(xr + hr)
        z = jax.nn.sigmoid(xz + hz)
        n = jnp.tanh(xn + r * hn)
        h_new = (1.0 - z) * n + z * h
        return h_new, h_new
    h0 = jnp.zeros((x.shape[0], Whh.shape[1]), x.dtype)
    _, ys = jax.lax.scan(step, h0, jnp.swapaxes(x, 0, 1))
    return jnp.swapaxes(ys, 0, 1)


def _pad_edges(idx, epad, fill):
    pad = jnp.full((epad - idx.shape[0],), fill, jnp.int32)
    return jnp.concatenate([idx.astype(jnp.int32), pad]).reshape(-1, BLK)


def kernel(low_x, z_std, land_std, edge_src_l2h, edge_dst_l2h, edge_index_high,
           gru_Wih, gru_Whh, gru_bih, gru_bhh, dense_W, dense_b, dbn_g, dbn_b,
           gc_Wrel, gc_brel, gc_Wroot, gat_Wl, gat_Wr, gat_att, gat_b, bn_g, bn_b,
           pred_W, pred_b):
    f32 = jnp.float32
    zeros16 = jnp.zeros((NPAD, 16), f32)
    zeros8 = jnp.zeros((NPAD, 8), f32)
    ones8 = jnp.concatenate(
        [jnp.ones((BLK, 1), f32), jnp.zeros((BLK, 7), f32)], axis=1)

    enc = _gru(low_x, gru_Wih, gru_Whh, gru_bih, gru_bhh).reshape(N_LOW, -1)
    enc = jnp.maximum(enc @ dense_W.T + dense_b, 0.0)
    enc = _batchnorm(enc, dbn_g, dbn_b)

    encw = enc @ gc_Wrel.T
    srcb_l = _pad_edges(edge_src_l2h, EPAD_L2H, 0)
    dstb_l = _pad_edges(edge_dst_l2h, EPAD_L2H, N_HIGH)
    part16, part8 = _l2h_call(encw, srcb_l, dstb_l, ones8, zeros16, zeros8)
    tot = (part16[0] + part16[1])[:N_HIGH]
    cnt = (part8[0] + part8[1])[:N_HIGH, 0]
    x_zland = jnp.concatenate([z_std, land_std], axis=-1)
    h = (tot / jnp.maximum(cnt, 1.0)[:, None]
         + gc_brel + x_zland @ gc_Wroot.T)

    srcb = _pad_edges(edge_index_high[0], EPAD_HH, 0)
    dstb = _pad_edges(edge_index_high[1], EPAD_HH, N_HIGH)
    pad16 = jnp.zeros((NPAD - N_HIGH, 16), f32)
    m8init = jnp.concatenate(
        [jnp.zeros((KB * BLK, 1), f32), jnp.ones((KB * BLK, 1), f32),
         jnp.zeros((KB * BLK, 6), f32)], axis=1)
    h = _batchnorm(h, bn_g[0], bn_b[0])
    for i in range(5):
        xl = h @ gat_Wl[i].T
        xr = h @ gat_Wr[i].T
        att = gat_att[i]
        part16, part8 = _gat_fused_call(
            jnp.concatenate([xl, pad16]), jnp.concatenate([xr, pad16]),
            srcb, dstb, att, m8init, zeros16, zeros8)
        tot = (part16[0] + part16[1])[:N_HIGH]
        aux = (part8[0] + part8[1])[:N_HIGH]
        ss = xl + xr
        ts = jnp.maximum(ss, 0.2 * ss)
        ees = jnp.exp(ts @ att)
        accv = tot + ees[:, None] * xl
        den = aux[:, 0] + ees
        cnt = aux[:, 1] + 1.0
        agg = accv / jnp.maximum(den, 1e-16)[:, None] / jnp.maximum(cnt, 1.0)[:, None]
        h = agg + gat_b[i]
        if i < 4:
            h = jnp.maximum(_batchnorm(h, bn_g[i + 1], bn_b[i + 1]), 0.0)
        else:
            h = jnp.maximum(h, 0.0)

    return h @ pred_W.T + pred_b


# speedup vs baseline: 44.8032x; 1.1313x over previous
import jax
import jax.numpy as jnp
from jax import lax
from jax.experimental import pallas as pl
from jax.experimental.pallas import tpu as pltpu
from jax.experimental.pallas import tpu_sc as plsc

N_LOW = 10000
N_HIGH = 50000
E_L2H = 800000
E_HH = 1600000
EPS = 1e-5

NC = 2
NS = 16
NW = NC * NS
BLK = 128
KB = 8
NPAD = 50048
NPT = NPAD // NS

EPAD_HH = 1638400
EPAD_L2H = 819200

_mesh = plsc.VectorSubcoreMesh(core_axis_name="c", subcore_axis_name="s",
                               num_cores=NC, num_subcores=NS)
_cparams = pltpu.CompilerParams(use_tc_tiling_on_sc=False, needs_layout_passes=False)


def _zero_accs(zeros16, zeros8, acc, acc2, s):
    pltpu.sync_copy(zeros16.at[pl.ds(s * NPT, NPT)], acc.at[pl.ds(s * NPT, NPT)])
    pltpu.sync_copy(zeros8.at[pl.ds(s * NPT, NPT)], acc2.at[pl.ds(s * NPT, NPT)])


def _dump_accs(acc, acc2, out, out2, c, s):
    pltpu.sync_copy(acc.at[pl.ds(s * NPT, NPT)], out.at[c, pl.ds(s * NPT, NPT)])
    pltpu.sync_copy(acc2.at[pl.ds(s * NPT, NPT)], out2.at[c, pl.ds(s * NPT, NPT)])


def _l2h_body(table, srcb, dstb, ones8, zeros16, zeros8, out, out2,
              acc, acc2, idxs, idxd, gbuf, obuf, gsem, ssem):
    c = lax.axis_index("c")
    s = lax.axis_index("s")
    w = s * NC + c
    bpw = EPAD_L2H // BLK // NW
    _zero_accs(zeros16, zeros8, acc, acc2, s)
    pltpu.sync_copy(ones8, obuf)
    plsc.subcore_barrier()

    def step(i, carry):
        base = w * bpw + i * KB
        pltpu.sync_copy(srcb.at[pl.ds(base, KB)], idxs)
        pltpu.sync_copy(dstb.at[pl.ds(base, KB)], idxd)
        gds = [pltpu.async_copy(table.at[idxs.at[j]], gbuf.at[j], gsem)
               for j in range(KB)]
        for d in gds:
            d.wait()
        sds = [pltpu.async_copy(gbuf.at[j], acc.at[idxd.at[j]], ssem, add=True)
               for j in range(KB)]
        sds += [pltpu.async_copy(obuf, acc2.at[idxd.at[j]], ssem, add=True)
                for j in range(KB)]
        for d in sds:
            d.wait()
        return carry

    lax.fori_loop(0, bpw // KB, step, 0)
    plsc.subcore_barrier()
    _dump_accs(acc, acc2, out, out2, c, s)


def _gat_fused_body(xl, xr, srcb, dstb, att, m8init, zeros16, zeros8,
                    out, out2, acc, acc2, idxs, idxd, bufa, bufb, m8buf,
                    attbuf, gsem, ssem):
    c = lax.axis_index("c")
    s = lax.axis_index("s")
    w = s * NC + c
    bpw = EPAD_HH // BLK // NW
    _zero_accs(zeros16, zeros8, acc, acc2, s)
    pltpu.sync_copy(att, attbuf)
    pltpu.sync_copy(m8init, m8buf)
    plsc.subcore_barrier()
    iota16 = lax.iota(jnp.int32, 16)
    attv = attbuf[...]
    att_s = [attv[k] for k in range(16)]
    kf = [jnp.full((16,), k, jnp.int32) for k in range(16)]
    zci = jnp.zeros((16,), jnp.int32)

    def grp(g, cc):
        lanes = g * 16 + iota16
        a_ks = []
        p0 = jnp.zeros((16,), jnp.float32)
        p1 = jnp.zeros((16,), jnp.float32)
        p2 = jnp.zeros((16,), jnp.float32)
        p3 = jnp.zeros((16,), jnp.float32)
        parts = [p0, p1, p2, p3]
        for k in range(16):
            a = plsc.load_gather(bufa, [lanes, kf[k]])
            b = plsc.load_gather(bufb, [lanes, kf[k]])
            a_ks.append(a)
            sk = a + b
            tk = jnp.maximum(sk, 0.2 * sk)
            parts[k % 4] = parts[k % 4] + tk * att_s[k]
        ee = jnp.exp((parts[0] + parts[1]) + (parts[2] + parts[3]))
        plsc.store_scatter(m8buf, [lanes, zci], ee)
        for k in range(16):
            plsc.store_scatter(bufa, [lanes, kf[k]], ee * a_ks[k])
        return cc

    def step(i, carry):
        base = w * bpw + i * KB
        pltpu.sync_copy(srcb.at[pl.ds(base, KB)], idxs)
        pltpu.sync_copy(dstb.at[pl.ds(base, KB)], idxd)
        gds = [(pltpu.async_copy(xl.at[idxs.at[j]],
                                 bufa.at[pl.ds(j * BLK, BLK)], gsem),
                pltpu.async_copy(xr.at[idxd.at[j]],
                                 bufb.at[pl.ds(j * BLK, BLK)], gsem))
               for j in range(KB)]
        sds = []
        for j in range(KB):
            gds[j][0].wait()
            gds[j][1].wait()
            lax.fori_loop(j * 8, (j + 1) * 8, grp, 0)
            sds.append(pltpu.async_copy(bufa.at[pl.ds(j * BLK, BLK)],
                                        acc.at[idxd.at[j]], ssem, add=True))
            sds.append(pltpu.async_copy(m8buf.at[pl.ds(j * BLK, BLK)],
                                        acc2.at[idxd.at[j]], ssem, add=True))
        for d in sds:
            d.wait()
        return carry

    lax.fori_loop(0, bpw // KB, step, 0)
    plsc.subcore_barrier()
    _dump_accs(acc, acc2, out, out2, c, s)


_l2h_call = pl.kernel(
    _l2h_body,
    out_type=[jax.ShapeDtypeStruct((NC, NPAD, 16), jnp.float32),
              jax.ShapeDtypeStruct((NC, NPAD, 8), jnp.float32)],
    mesh=_mesh,
    compiler_params=_cparams,
    scratch_types=[
        pltpu.VMEM_SHARED((NPAD, 16), jnp.float32),
        pltpu.VMEM_SHARED((NPAD, 8), jnp.float32),
        pltpu.VMEM((KB, BLK), jnp.int32),
        pltpu.VMEM((KB, BLK), jnp.int32),
        pltpu.VMEM((KB, BLK, 16), jnp.float32),
        pltpu.VMEM((BLK, 8), jnp.float32),
        pltpu.SemaphoreType.DMA,
        pltpu.SemaphoreType.DMA,
    ],
)

_gat_fused_call = pl.kernel(
    _gat_fused_body,
    out_type=[jax.ShapeDtypeStruct((NC, NPAD, 16), jnp.float32),
              jax.ShapeDtypeStruct((NC, NPAD, 8), jnp.float32)],
    mesh=_mesh,
    compiler_params=_cparams,
    scratch_types=[
        pltpu.VMEM_SHARED((NPAD, 16), jnp.float32),
        pltpu.VMEM_SHARED((NPAD, 8), jnp.float32),
        pltpu.VMEM((KB, BLK), jnp.int32),
        pltpu.VMEM((KB, BLK), jnp.int32),
        pltpu.VMEM((KB * BLK, 16), jnp.float32),
        pltpu.VMEM((KB * BLK, 16), jnp.float32),
        pltpu.VMEM((KB * BLK, 8), jnp.float32),
        pltpu.VMEM((16,), jnp.float32),
        pltpu.SemaphoreType.DMA,
        pltpu.SemaphoreType.DMA,
    ],
)


BR = 3128
NGRID = NPAD // BR
BRF = 2000


def _stats(h, r):
    row = r * BR + lax.broadcasted_iota(jnp.int32, (BR, 1), 0)
    mask = row < N_HIGH
    hm = jnp.where(mask, h, 0.0)
    return jnp.concatenate([jnp.sum(hm, axis=0, keepdims=True),
                            jnp.sum(hm * hm, axis=0, keepdims=True)], axis=0)


def _init_pre_body(part16, part8, xzl, wroot, brel, o_h, o_st):
    tot = part16[0] + part16[1]
    cnt = part8[0, :, 0:1] + part8[1, :, 0:1]
    h = (tot / jnp.maximum(cnt, 1.0) + brel[...]
         + jnp.dot(xzl[...], wroot[...].T, preferred_element_type=jnp.float32))
    o_h[...] = h
    o_st[0] = _stats(h, pl.program_id(0))


def _gat_pre_body(part16, part8, xl, xr, att, bias, o_h, o_st):
    tot = part16[0] + part16[1]
    xlv = xl[...]
    xrv = xr[...]
    ss = xlv + xrv
    ts = jnp.maximum(ss, 0.2 * ss)
    es = jnp.sum(ts * att[...], axis=1, keepdims=True)
    ees = jnp.exp(es)
    accv = tot + ees * xlv
    den = part8[0, :, 0:1] + part8[1, :, 0:1] + ees
    cnt = part8[0, :, 1:2] + part8[1, :, 1:2] + 1.0
    agg = accv / jnp.maximum(den, 1e-16) / jnp.maximum(cnt, 1.0)
    h = agg + bias[...]
    o_h[...] = h
    o_st[0] = _stats(h, pl.program_id(0))


def _make_bn_mm_body(relu):
    def body(hp, scale, shift, wl, wr, o_xl, o_xr):
        h = hp[...] * scale[...] + shift[...]
        if relu:
            h = jnp.maximum(h, 0.0)
        o_xl[...] = jnp.dot(h, wl[...].T, preferred_element_type=jnp.float32)
        o_xr[...] = jnp.dot(h, wr[...].T, preferred_element_type=jnp.float32)
    return body


def _final_body(part16, part8, xl, xr, att, bias, pw, pb, o):
    tot = part16[0] + part16[1]
    xlv = xl[...]
    xrv = xr[...]
    ss = xlv + xrv
    ts = jnp.maximum(ss, 0.2 * ss)
    es = jnp.sum(ts * att[...], axis=1, keepdims=True)
    ees = jnp.exp(es)
    accv = tot + ees * xlv
    den = part8[0, :, 0:1] + part8[1, :, 0:1] + ees
    cnt = part8[0, :, 1:2] + part8[1, :, 1:2] + 1.0
    agg = accv / jnp.maximum(den, 1e-16) / jnp.maximum(cnt, 1.0)
    h = jnp.maximum(agg + bias[...], 0.0)
    o[...] = jnp.sum(h * pw[...], axis=1, keepdims=True) + pb[...]


def _bspec(bshape, f=None):
    return pl.BlockSpec(bshape, f if f is not None else (lambda r: (0, 0)))


_p16_spec = pl.BlockSpec((NC, BR, 16), lambda r: (0, r, 0))
_p8_spec = pl.BlockSpec((NC, BR, 8), lambda r: (0, r, 0))
_row_spec = pl.BlockSpec((BR, 16), lambda r: (r, 0))

_init_pre_call = pl.pallas_call(
    _init_pre_body,
    grid=(NGRID,),
    in_specs=[_p16_spec, _p8_spec, pl.BlockSpec((BR, 7), lambda r: (r, 0)),
              _bspec((16, 7)), _bspec((1, 16))],
    out_specs=[_row_spec, pl.BlockSpec((1, 2, 16), lambda r: (r, 0, 0))],
    out_shape=[jax.ShapeDtypeStruct((NPAD, 16), jnp.float32),
               jax.ShapeDtypeStruct((NGRID, 2, 16), jnp.float32)])

_gat_pre_call = pl.pallas_call(
    _gat_pre_body,
    grid=(NGRID,),
    in_specs=[_p16_spec, _p8_spec, _row_spec, _row_spec,
              _bspec((1, 16)), _bspec((1, 16))],
    out_specs=[_row_spec, pl.BlockSpec((1, 2, 16), lambda r: (r, 0, 0))],
    out_shape=[jax.ShapeDtypeStruct((NPAD, 16), jnp.float32),
               jax.ShapeDtypeStruct((NGRID, 2, 16), jnp.float32)])

_bn_mm_calls = {
    relu: pl.pallas_call(
        _make_bn_mm_body(relu),
        grid=(NGRID,),
        in_specs=[_row_spec, _bspec((1, 16)), _bspec((1, 16)),
                  _bspec((16, 16)), _bspec((16, 16))],
        out_specs=[_row_spec, _row_spec],
        out_shape=[jax.ShapeDtypeStruct((NPAD, 16), jnp.float32),
                   jax.ShapeDtypeStruct((NPAD, 16), jnp.float32)])
    for relu in (False, True)}

_final_call = pl.pallas_call(
    _final_body,
    grid=(N_HIGH // BRF,),
    in_specs=[pl.BlockSpec((NC, BRF, 16), lambda r: (0, r, 0)),
              pl.BlockSpec((NC, BRF, 8), lambda r: (0, r, 0)),
              pl.BlockSpec((BRF, 16), lambda r: (r, 0)),
              pl.BlockSpec((BRF, 16), lambda r: (r, 0)),
              _bspec((1, 16)), _bspec((1, 16)), _bspec((1, 16)),
              _bspec((1, 1))],
    out_specs=pl.BlockSpec((BRF, 1), lambda r: (r, 0)),
    out_shape=jax.ShapeDtypeStruct((N_HIGH, 1), jnp.float32))


def _bn_scale_shift(st, g, b):
    tot = jnp.sum(st, axis=0)
    m = tot[0] / N_HIGH
    v = tot[1] / N_HIGH - m * m
    scale = g / jnp.sqrt(v + EPS)
    return scale[None], (b - m * scale)[None]


def _batchnorm(x, g, b):
    m = x.mean(0)
    v = x.var(0)
    return (x - m) / jnp.sqrt(v + EPS) * g + b


def _gru(x, Wih, Whh, bih, bhh):
    def step(h, xt):
        gx = xt @ Wih.T + bih
        gh = h @ Whh.T + bhh
        xr, xz, xn = jnp.split(gx, 3, axis=-1)
        hr, hz, hn = jnp.split(gh, 3, axis=-1)
        r = jax.nn.sigmoid(xr + hr)
        z = jax.nn.sigmoid(xz + hz)
        n = jnp.tanh(xn + r * hn)
        h_new = (1.0 - z) * n + z * h
        return h_new, h_new
    h0 = jnp.zeros((x.shape[0], Whh.shape[1]), x.dtype)
    _, ys = jax.lax.scan(step, h0, jnp.swapaxes(x, 0, 1))
    return jnp.swapaxes(ys, 0, 1)


def _pad_edges(idx, epad, fill):
    n = epad - idx.shape[0]
    if fill == 0:
        pad = jnp.zeros((n,), jnp.int32)
    else:
        pad = N_HIGH + (jnp.arange(n, dtype=jnp.int32) % (NPAD - N_HIGH))
    return jnp.concatenate([idx.astype(jnp.int32), pad]).reshape(-1, BLK)


def kernel(low_x, z_std, land_std, edge_src_l2h, edge_dst_l2h, edge_index_high,
           gru_Wih, gru_Whh, gru_bih, gru_bhh, dense_W, dense_b, dbn_g, dbn_b,
           gc_Wrel, gc_brel, gc_Wroot, gat_Wl, gat_Wr, gat_att, gat_b, bn_g, bn_b,
           pred_W, pred_b):
    f32 = jnp.float32
    zeros16 = jnp.zeros((NPAD, 16), f32)
    zeros8 = jnp.zeros((NPAD, 8), f32)
    ones8 = jnp.concatenate(
        [jnp.ones((BLK, 1), f32), jnp.zeros((BLK, 7), f32)], axis=1)

    enc = _gru(low_x, gru_Wih, gru_Whh, gru_bih, gru_bhh).reshape(N_LOW, -1)
    enc = jnp.maximum(enc @ dense_W.T + dense_b, 0.0)
    enc = _batchnorm(enc, dbn_g, dbn_b)

    encw = enc @ gc_Wrel.T
    srcb_l = _pad_edges(edge_src_l2h, EPAD_L2H, 0)
    dstb_l = _pad_edges(edge_dst_l2h, EPAD_L2H, N_HIGH)
    part16, part8 = _l2h_call(encw, srcb_l, dstb_l, ones8, zeros16, zeros8)
    xzl = jnp.concatenate([z_std, land_std], axis=-1)
    xzl = jnp.concatenate([xzl, jnp.zeros((NPAD - N_HIGH, 7), f32)], axis=0)

    srcb = _pad_edges(edge_index_high[0], EPAD_HH, 0)
    dstb = _pad_edges(edge_index_high[1], EPAD_HH, N_HIGH)
    m8init = jnp.concatenate(
        [jnp.zeros((KB * BLK, 1), f32), jnp.ones((KB * BLK, 1), f32),
         jnp.zeros((KB * BLK, 6), f32)], axis=1)
    hp, st = _init_pre_call(part16, part8, xzl, gc_Wroot, gc_brel[None])
    scale, shift = _bn_scale_shift(st, bn_g[0], bn_b[0])
    xl, xr = _bn_mm_calls[False](hp, scale, shift, gat_Wl[0], gat_Wr[0])
    for i in range(5):
        p16, p8 = _gat_fused_call(xl, xr, srcb, dstb, gat_att[i], m8init,
                                  zeros16, zeros8)
        if i < 4:
            hp, st = _gat_pre_call(p16, p8, xl, xr, gat_att[i][None],
                                   gat_b[i][None])
            scale, shift = _bn_scale_shift(st, bn_g[i + 1], bn_b[i + 1])
            xl, xr = _bn_mm_calls[True](hp, scale, shift,
                                        gat_Wl[i + 1], gat_Wr[i + 1])
        else:
            out = _final_call(p16, p8, xl, xr, gat_att[i][None],
                              gat_b[i][None], pred_W, pred_b[None])
    return out


# speedup vs baseline: 44.9027x; 1.0022x over previous
import jax
import jax.numpy as jnp
from jax import lax
from jax.experimental import pallas as pl
from jax.experimental.pallas import tpu as pltpu
from jax.experimental.pallas import tpu_sc as plsc

N_LOW = 10000
N_HIGH = 50000
E_L2H = 800000
E_HH = 1600000
EPS = 1e-5

NC = 2
NS = 16
NW = NC * NS
BLK = 128
KB = 8
NPAD = 50048
NPT = NPAD // NS

EPAD_HH = 1638400
EPAD_L2H = 819200

_mesh = plsc.VectorSubcoreMesh(core_axis_name="c", subcore_axis_name="s",
                               num_cores=NC, num_subcores=NS)
_cparams = pltpu.CompilerParams(use_tc_tiling_on_sc=False, needs_layout_passes=False)


def _zero_accs(zeros16, zeros8, acc, acc2, s):
    pltpu.sync_copy(zeros16.at[pl.ds(s * NPT, NPT)], acc.at[pl.ds(s * NPT, NPT)])
    pltpu.sync_copy(zeros8.at[pl.ds(s * NPT, NPT)], acc2.at[pl.ds(s * NPT, NPT)])


def _dump_accs(acc, acc2, out, out2, c, s):
    pltpu.sync_copy(acc.at[pl.ds(s * NPT, NPT)], out.at[c, pl.ds(s * NPT, NPT)])
    pltpu.sync_copy(acc2.at[pl.ds(s * NPT, NPT)], out2.at[c, pl.ds(s * NPT, NPT)])


def _l2h_body(table, srcb, dstb, ones8, zeros16, zeros8, out, out2,
              acc, acc2, idxs, idxd, gbuf, obuf, gsem, ssem):
    c = lax.axis_index("c")
    s = lax.axis_index("s")
    w = s * NC + c
    bpw = EPAD_L2H // BLK // NW
    _zero_accs(zeros16, zeros8, acc, acc2, s)
    pltpu.sync_copy(ones8, obuf)
    plsc.subcore_barrier()

    def step(i, carry):
        base = w * bpw + i * KB
        pltpu.sync_copy(srcb.at[pl.ds(base, KB)], idxs)
        pltpu.sync_copy(dstb.at[pl.ds(base, KB)], idxd)
        gds = [pltpu.async_copy(table.at[idxs.at[j]], gbuf.at[j], gsem)
               for j in range(KB)]
        sds = []
        for j in range(KB):
            gds[j].wait()
            sds.append(pltpu.async_copy(gbuf.at[j], acc.at[idxd.at[j]],
                                        ssem, add=True))
            sds.append(pltpu.async_copy(obuf, acc2.at[idxd.at[j]],
                                        ssem, add=True))
        for d in sds:
            d.wait()
        return carry

    lax.fori_loop(0, bpw // KB, step, 0)
    plsc.subcore_barrier()
    _dump_accs(acc, acc2, out, out2, c, s)


def _gat_fused_body(xl, xr, srcb, dstb, att, m8init, zeros16, zeros8,
                    out, out2, acc, acc2, idxs, idxd, bufa, bufb, m8buf,
                    attbuf, gsem, ssem):
    c = lax.axis_index("c")
    s = lax.axis_index("s")
    w = s * NC + c
    bpw = EPAD_HH // BLK // NW
    _zero_accs(zeros16, zeros8, acc, acc2, s)
    pltpu.sync_copy(att, attbuf)
    pltpu.sync_copy(m8init, m8buf)
    plsc.subcore_barrier()
    iota16 = lax.iota(jnp.int32, 16)
    attv = attbuf[...]
    att_s = [attv[k] for k in range(16)]
    kf = [jnp.full((16,), k, jnp.int32) for k in range(16)]
    zci = jnp.zeros((16,), jnp.int32)

    def grp(g, cc):
        lanes = g * 16 + iota16
        a_ks = []
        p0 = jnp.zeros((16,), jnp.float32)
        p1 = jnp.zeros((16,), jnp.float32)
        p2 = jnp.zeros((16,), jnp.float32)
        p3 = jnp.zeros((16,), jnp.float32)
        parts = [p0, p1, p2, p3]
        for k in range(16):
            a = plsc.load_gather(bufa, [lanes, kf[k]])
            b = plsc.load_gather(bufb, [lanes, kf[k]])
            a_ks.append(a)
            sk = a + b
            tk = jnp.maximum(sk, 0.2 * sk)
            parts[k % 4] = parts[k % 4] + tk * att_s[k]
        ee = jnp.exp((parts[0] + parts[1]) + (parts[2] + parts[3]))
        plsc.store_scatter(m8buf, [lanes, zci], ee)
        for k in range(16):
            plsc.store_scatter(bufa, [lanes, kf[k]], ee * a_ks[k])
        return cc

    def step(i, carry):
        base = w * bpw + i * KB
        pltpu.sync_copy(srcb.at[pl.ds(base, KB)], idxs)
        pltpu.sync_copy(dstb.at[pl.ds(base, KB)], idxd)
        gds = [(pltpu.async_copy(xl.at[idxs.at[j]],
                                 bufa.at[pl.ds(j * BLK, BLK)], gsem),
                pltpu.async_copy(xr.at[idxd.at[j]],
                                 bufb.at[pl.ds(j * BLK, BLK)], gsem))
               for j in range(KB)]
        sds = []
        for j in range(KB):
            gds[j][0].wait()
            gds[j][1].wait()
            lax.fori_loop(j * 8, (j + 1) * 8, grp, 0)
            sds.append(pltpu.async_copy(bufa.at[pl.ds(j * BLK, BLK)],
                                        acc.at[idxd.at[j]], ssem, add=True))
            sds.append(pltpu.async_copy(m8buf.at[pl.ds(j * BLK, BLK)],
                                        acc2.at[idxd.at[j]], ssem, add=True))
        for d in sds:
            d.wait()
        return carry

    lax.fori_loop(0, bpw // KB, step, 0)
    plsc.subcore_barrier()
    _dump_accs(acc, acc2, out, out2, c, s)


_l2h_call = pl.kernel(
    _l2h_body,
    out_type=[jax.ShapeDtypeStruct((NC, NPAD, 16), jnp.float32),
              jax.ShapeDtypeStruct((NC, NPAD, 8), jnp.float32)],
    mesh=_mesh,
    compiler_params=_cparams,
    scratch_types=[
        pltpu.VMEM_SHARED((NPAD, 16), jnp.float32),
        pltpu.VMEM_SHARED((NPAD, 8), jnp.float32),
        pltpu.VMEM((KB, BLK), jnp.int32),
        pltpu.VMEM((KB, BLK), jnp.int32),
        pltpu.VMEM((KB, BLK, 16), jnp.float32),
        pltpu.VMEM((BLK, 8), jnp.float32),
        pltpu.SemaphoreType.DMA,
        pltpu.SemaphoreType.DMA,
    ],
)

_gat_fused_call = pl.kernel(
    _gat_fused_body,
    out_type=[jax.ShapeDtypeStruct((NC, NPAD, 16), jnp.float32),
              jax.ShapeDtypeStruct((NC, NPAD, 8), jnp.float32)],
    mesh=_mesh,
    compiler_params=_cparams,
    scratch_types=[
        pltpu.VMEM_SHARED((NPAD, 16), jnp.float32),
        pltpu.VMEM_SHARED((NPAD, 8), jnp.float32),
        pltpu.VMEM((KB, BLK), jnp.int32),
        pltpu.VMEM((KB, BLK), jnp.int32),
        pltpu.VMEM((KB * BLK, 16), jnp.float32),
        pltpu.VMEM((KB * BLK, 16), jnp.float32),
        pltpu.VMEM((KB * BLK, 8), jnp.float32),
        pltpu.VMEM((16,), jnp.float32),
        pltpu.SemaphoreType.DMA,
        pltpu.SemaphoreType.DMA,
    ],
)


BR = 3128
NGRID = NPAD // BR
BRF = 2000


def _stats(h, r):
    row = r * BR + lax.broadcasted_iota(jnp.int32, (BR, 1), 0)
    mask = row < N_HIGH
    hm = jnp.where(mask, h, 0.0)
    return jnp.concatenate([jnp.sum(hm, axis=0, keepdims=True),
                            jnp.sum(hm * hm, axis=0, keepdims=True)], axis=0)


def _init_pre_body(part16, part8, xzl, wroot, brel, o_h, o_st):
    tot = part16[0] + part16[1]
    cnt = part8[0, :, 0:1] + part8[1, :, 0:1]
    h = (tot / jnp.maximum(cnt, 1.0) + brel[...]
         + jnp.dot(xzl[...], wroot[...].T, preferred_element_type=jnp.float32))
    o_h[...] = h
    o_st[0] = _stats(h, pl.program_id(0))


def _gat_pre_body(part16, part8, xl, xr, att, bias, o_h, o_st):
    tot = part16[0] + part16[1]
    xlv = xl[...]
    xrv = xr[...]
    ss = xlv + xrv
    ts = jnp.maximum(ss, 0.2 * ss)
    es = jnp.sum(ts * att[...], axis=1, keepdims=True)
    ees = jnp.exp(es)
    accv = tot + ees * xlv
    den = part8[0, :, 0:1] + part8[1, :, 0:1] + ees
    cnt = part8[0, :, 1:2] + part8[1, :, 1:2] + 1.0
    agg = accv / jnp.maximum(den, 1e-16) / jnp.maximum(cnt, 1.0)
    h = agg + bias[...]
    o_h[...] = h
    o_st[0] = _stats(h, pl.program_id(0))


def _make_bn_mm_body(relu):
    def body(hp, scale, shift, wl, wr, o_xl, o_xr):
        h = hp[...] * scale[...] + shift[...]
        if relu:
            h = jnp.maximum(h, 0.0)
        o_xl[...] = jnp.dot(h, wl[...].T, preferred_element_type=jnp.float32)
        o_xr[...] = jnp.dot(h, wr[...].T, preferred_element_type=jnp.float32)
    return body


def _final_body(part16, part8, xl, xr, att, bias, pw, pb, o):
    tot = part16[0] + part16[1]
    xlv = xl[...]
    xrv = xr[...]
    ss = xlv + xrv
    ts = jnp.maximum(ss, 0.2 * ss)
    es = jnp.sum(ts * att[...], axis=1, keepdims=True)
    ees = jnp.exp(es)
    accv = tot + ees * xlv
    den = part8[0, :, 0:1] + part8[1, :, 0:1] + ees
    cnt = part8[0, :, 1:2] + part8[1, :, 1:2] + 1.0
    agg = accv / jnp.maximum(den, 1e-16) / jnp.maximum(cnt, 1.0)
    h = jnp.maximum(agg + bias[...], 0.0)
    o[...] = jnp.sum(h * pw[...], axis=1, keepdims=True) + pb[...]


def _bspec(bshape, f=None):
    return pl.BlockSpec(bshape, f if f is not None else (lambda r: (0, 0)))


_p16_spec = pl.BlockSpec((NC, BR, 16), lambda r: (0, r, 0))
_p8_spec = pl.BlockSpec((NC, BR, 8), lambda r: (0, r, 0))
_row_spec = pl.BlockSpec((BR, 16), lambda r: (r, 0))

_init_pre_call = pl.pallas_call(
    _init_pre_body,
    grid=(NGRID,),
    in_specs=[_p16_spec, _p8_spec, pl.BlockSpec((BR, 7), lambda r: (r, 0)),
              _bspec((16, 7)), _bspec((1, 16))],
    out_specs=[_row_spec, pl.BlockSpec((1, 2, 16), lambda r: (r, 0, 0))],
    out_shape=[jax.ShapeDtypeStruct((NPAD, 16), jnp.float32),
               jax.ShapeDtypeStruct((NGRID, 2, 16), jnp.float32)])

_gat_pre_call = pl.pallas_call(
    _gat_pre_body,
    grid=(NGRID,),
    in_specs=[_p16_spec, _p8_spec, _row_spec, _row_spec,
              _bspec((1, 16)), _bspec((1, 16))],
    out_specs=[_row_spec, pl.BlockSpec((1, 2, 16), lambda r: (r, 0, 0))],
    out_shape=[jax.ShapeDtypeStruct((NPAD, 16), jnp.float32),
               jax.ShapeDtypeStruct((NGRID, 2, 16), jnp.float32)])

_bn_mm_calls = {
    relu: pl.pallas_call(
        _make_bn_mm_body(relu),
        grid=(NGRID,),
        in_specs=[_row_spec, _bspec((1, 16)), _bspec((1, 16)),
                  _bspec((16, 16)), _bspec((16, 16))],
        out_specs=[_row_spec, _row_spec],
        out_shape=[jax.ShapeDtypeStruct((NPAD, 16), jnp.float32),
                   jax.ShapeDtypeStruct((NPAD, 16), jnp.float32)])
    for relu in (False, True)}

_final_call = pl.pallas_call(
    _final_body,
    grid=(N_HIGH // BRF,),
    in_specs=[pl.BlockSpec((NC, BRF, 16), lambda r: (0, r, 0)),
              pl.BlockSpec((NC, BRF, 8), lambda r: (0, r, 0)),
              pl.BlockSpec((BRF, 16), lambda r: (r, 0)),
              pl.BlockSpec((BRF, 16), lambda r: (r, 0)),
              _bspec((1, 16)), _bspec((1, 16)), _bspec((1, 16)),
              _bspec((1, 1))],
    out_specs=pl.BlockSpec((BRF, 1), lambda r: (r, 0)),
    out_shape=jax.ShapeDtypeStruct((N_HIGH, 1), jnp.float32))


def _bn_scale_shift(st, g, b):
    tot = jnp.sum(st, axis=0)
    m = tot[0] / N_HIGH
    v = tot[1] / N_HIGH - m * m
    scale = g / jnp.sqrt(v + EPS)
    return scale[None], (b - m * scale)[None]


def _batchnorm(x, g, b):
    m = x.mean(0)
    v = x.var(0)
    return (x - m) / jnp.sqrt(v + EPS) * g + b


def _gru(x, Wih, Whh, bih, bhh):
    def step(h, xt):
        gx = xt @ Wih.T + bih
        gh = h @ Whh.T + bhh
        xr, xz, xn = jnp.split(gx, 3, axis=-1)
        hr, hz, hn = jnp.split(gh, 3, axis=-1)
        r = jax.nn.sigmoid(xr + hr)
        z = jax.nn.sigmoid(xz + hz)
        n = jnp.tanh(xn + r * hn)
        h_new = (1.0 - z) * n + z * h
        return h_new, h_new
    h0 = jnp.zeros((x.shape[0], Whh.shape[1]), x.dtype)
    _, ys = jax.lax.scan(step, h0, jnp.swapaxes(x, 0, 1))
    return jnp.swapaxes(ys, 0, 1)


def _pad_edges(idx, epad, fill):
    n = epad - idx.shape[0]
    if fill == 0:
        pad = jnp.zeros((n,), jnp.int32)
    else:
        pad = N_HIGH + (jnp.arange(n, dtype=jnp.int32) % (NPAD - N_HIGH))
    return jnp.concatenate([idx.astype(jnp.int32), pad]).reshape(-1, BLK)


def kernel(low_x, z_std, land_std, edge_src_l2h, edge_dst_l2h, edge_index_high,
           gru_Wih, gru_Whh, gru_bih, gru_bhh, dense_W, dense_b, dbn_g, dbn_b,
           gc_Wrel, gc_brel, gc_Wroot, gat_Wl, gat_Wr, gat_att, gat_b, bn_g, bn_b,
           pred_W, pred_b):
    f32 = jnp.float32
    zeros16 = jnp.zeros((NPAD, 16), f32)
    zeros8 = jnp.zeros((NPAD, 8), f32)
    ones8 = jnp.concatenate(
        [jnp.ones((BLK, 1), f32), jnp.zeros((BLK, 7), f32)], axis=1)

    enc = _gru(low_x, gru_Wih, gru_Whh, gru_bih, gru_bhh).reshape(N_LOW, -1)
    enc = jnp.maximum(enc @ dense_W.T + dense_b, 0.0)
    enc = _batchnorm(enc, dbn_g, dbn_b)

    encw = enc @ gc_Wrel.T
    srcb_l = _pad_edges(edge_src_l2h, EPAD_L2H, 0)
    dstb_l = _pad_edges(edge_dst_l2h, EPAD_L2H, N_HIGH)
    part16, part8 = _l2h_call(encw, srcb_l, dstb_l, ones8, zeros16, zeros8)
    xzl = jnp.concatenate([z_std, land_std], axis=-1)
    xzl = jnp.concatenate([xzl, jnp.zeros((NPAD - N_HIGH, 7), f32)], axis=0)

    srcb = _pad_edges(edge_index_high[0], EPAD_HH, 0)
    dstb = _pad_edges(edge_index_high[1], EPAD_HH, N_HIGH)
    m8init = jnp.concatenate(
        [jnp.zeros((KB * BLK, 1), f32), jnp.ones((KB * BLK, 1), f32),
         jnp.zeros((KB * BLK, 6), f32)], axis=1)
    hp, st = _init_pre_call(part16, part8, xzl, gc_Wroot, gc_brel[None])
    scale, shift = _bn_scale_shift(st, bn_g[0], bn_b[0])
    xl, xr = _bn_mm_calls[False](hp, scale, shift, gat_Wl[0], gat_Wr[0])
    for i in range(5):
        p16, p8 = _gat_fused_call(xl, xr, srcb, dstb, gat_att[i], m8init,
                                  zeros16, zeros8)
        if i < 4:
            hp, st = _gat_pre_call(p16, p8, xl, xr, gat_att[i][None],
                                   gat_b[i][None])
            scale, shift = _bn_scale_shift(st, bn_g[i + 1], bn_b[i + 1])
            xl, xr = _bn_mm_calls[True](hp, scale, shift,
                                        gat_Wl[i + 1], gat_Wr[i + 1])
        else:
            out = _final_call(p16, p8, xl, xr, gat_att[i][None],
                              gat_b[i][None], pred_W, pred_b[None])
    return out
